# Initial kernel scaffold; baseline (speedup 1.0000x reference)
import jax
import jax.numpy as jnp
from jax.experimental import pallas as pl


def kernel(tokens, token_types, word_table, type_table):
    raise NotImplementedError("write your pallas kernel here")


# speedup vs baseline: 5.1608x; 5.1608x over previous
import functools

import jax
import jax.numpy as jnp
from jax import lax
from jax.experimental import pallas as pl
from jax.experimental.pallas import tpu as pltpu
from jax.experimental.pallas import tpu_sc as plsc

VOCAB = 100000
DIM = 64
BATCH = 4096
SEQ = 200
NUM_TOK = BATCH * SEQ
NUM_WORKERS = 32
PER_W = NUM_TOK // NUM_WORKERS
CHUNK = 128
NCHUNKS = PER_W // CHUNK
UNROLL = 4

_mesh = plsc.VectorSubcoreMesh(core_axis_name="c", subcore_axis_name="s")


def _positional_encoding(token_length, embedding_dim):
    pos = jnp.arange(token_length, dtype=jnp.float32)[:, None]
    i = jnp.arange(embedding_dim)[None, :]
    angle_rates = 1.0 / jnp.power(
        10000.0, (2 * (i // 2)).astype(jnp.float32) / embedding_dim)
    angles = pos * angle_rates
    return jnp.where(i % 2 == 0, jnp.sin(angles), jnp.cos(angles)).astype(jnp.float32)


@functools.partial(
    pl.kernel,
    mesh=_mesh,
    out_type=jax.ShapeDtypeStruct((NUM_TOK, DIM), jnp.float32),
    scratch_types=[
        pltpu.VMEM((PER_W,), jnp.int32),
        pltpu.VMEM((PER_W,), jnp.int32),
        pltpu.VMEM((CHUNK, DIM), jnp.float32),
        pltpu.VMEM((CHUNK, DIM), jnp.float32),
        pltpu.SemaphoreType.DMA,
    ],
    compiler_params=pltpu.CompilerParams(use_tc_tiling_on_sc=False),
)
def _emb_kernel(widx_hbm, pidx_hbm, word_hbm, pet_hbm,
                out_hbm, widx_v, pidx_v, wrows, prows, sem):
    wid = lax.axis_index("s") * 2 + lax.axis_index("c")
    base = pl.multiple_of(wid * PER_W, PER_W)
    pltpu.sync_copy(widx_hbm.at[pl.ds(base, PER_W)], widx_v)
    pltpu.sync_copy(pidx_hbm.at[pl.ds(base, PER_W)], pidx_v)

    def chunk_body(it, carry):
        off = pl.multiple_of(it * CHUNK, CHUNK)
        cw = pltpu.async_copy(word_hbm.at[widx_v.at[pl.ds(off, CHUNK)]], wrows, sem)
        cp = pltpu.async_copy(pet_hbm.at[pidx_v.at[pl.ds(off, CHUNK)]], prows, sem)
        cw.wait()
        cp.wait()

        def row_body(r, c2):
            for u in range(UNROLL):
                for j in range(DIM // 16):
                    s = pl.ds(j * 16, 16)
                    i = r * UNROLL + u
                    wrows[i, s] = (wrows[i, s] + prows[i, s]) * 8.0
            return c2

        lax.fori_loop(0, CHUNK // UNROLL, row_body, 0)
        pltpu.sync_copy(wrows, out_hbm.at[pl.ds(base + off, CHUNK)])
        return carry

    lax.fori_loop(0, NCHUNKS, chunk_body, 0)


def kernel(tokens, token_types, word_table, type_table):
    widx = tokens.reshape(NUM_TOK).astype(jnp.int32)
    pos = jnp.arange(SEQ, dtype=jnp.int32)
    pidx = (token_types.astype(jnp.int32) * SEQ + pos[None, :]).reshape(NUM_TOK)
    pe = _positional_encoding(SEQ, DIM)
    pet = (pe[None, :, :] / 8.0 + type_table[:, None, :]).reshape(2 * SEQ, DIM)
    pet = pet.astype(jnp.float32)
    out = _emb_kernel(widx, pidx, word_table, pet)
    return out.reshape(BATCH, SEQ, DIM)


# speedup vs baseline: 5.4839x; 1.0626x over previous
import functools

import jax
import jax.numpy as jnp
from jax import lax
from jax.experimental import pallas as pl
from jax.experimental.pallas import tpu as pltpu
from jax.experimental.pallas import tpu_sc as plsc

VOCAB = 100000
DIM = 64
BATCH = 4096
SEQ = 200
NUM_TOK = BATCH * SEQ
NUM_WORKERS = 32
PER_W = NUM_TOK // NUM_WORKERS
CHUNK = 256
GHALF = 128
NCHUNKS = PER_W // CHUNK
UNROLL = 8

_mesh = plsc.VectorSubcoreMesh(core_axis_name="c", subcore_axis_name="s")


def _positional_encoding(token_length, embedding_dim):
    pos = jnp.arange(token_length, dtype=jnp.float32)[:, None]
    i = jnp.arange(embedding_dim)[None, :]
    angle_rates = 1.0 / jnp.power(
        10000.0, (2 * (i // 2)).astype(jnp.float32) / embedding_dim)
    angles = pos * angle_rates
    return jnp.where(i % 2 == 0, jnp.sin(angles), jnp.cos(angles)).astype(jnp.float32)


@functools.partial(
    pl.kernel,
    mesh=_mesh,
    out_type=jax.ShapeDtypeStruct((NUM_TOK, DIM), jnp.float32),
    scratch_types=[
        pltpu.VMEM((PER_W,), jnp.int32),
        pltpu.VMEM((PER_W,), jnp.int32),
        pltpu.VMEM((CHUNK, DIM), jnp.float32),
        pltpu.VMEM((CHUNK, DIM), jnp.float32),
        pltpu.VMEM((CHUNK, DIM), jnp.float32),
        pltpu.VMEM((CHUNK, DIM), jnp.float32),
        pltpu.SemaphoreType.DMA,
        pltpu.SemaphoreType.DMA,
        pltpu.SemaphoreType.DMA,
        pltpu.SemaphoreType.DMA,
    ],
    compiler_params=pltpu.CompilerParams(use_tc_tiling_on_sc=False),
)
def _emb_kernel(widx_hbm, pidx_hbm, word_hbm, pet_hbm, out_hbm,
                widx_v, pidx_v, w0, p0, w1, p1, g0, g1, wr0, wr1):
    wid = lax.axis_index("s") * 2 + lax.axis_index("c")
    base = pl.multiple_of(wid * PER_W, PER_W)
    pltpu.sync_copy(widx_hbm.at[pl.ds(base, PER_W)], widx_v)
    pltpu.sync_copy(pidx_hbm.at[pl.ds(base, PER_W)], pidx_v)

    wbufs, pbufs, gsems, wsems = (w0, w1), (p0, p1), (g0, g1), (wr0, wr1)

    def issue_gather(it, b):
        off = pl.multiple_of(it * CHUNK, CHUNK)
        for h in range(CHUNK // GHALF):
            ho = off + h * GHALF
            dsl = pl.ds(h * GHALF, GHALF)
            pltpu.async_copy(word_hbm.at[widx_v.at[pl.ds(ho, GHALF)]],
                             wbufs[b].at[dsl], gsems[b])
            pltpu.async_copy(pet_hbm.at[pidx_v.at[pl.ds(ho, GHALF)]],
                             pbufs[b].at[dsl], gsems[b])

    def wait_gather(b):
        dummy = out_hbm.at[pl.ds(0, CHUNK)]
        pltpu.make_async_copy(dummy, wbufs[b], gsems[b]).wait()
        pltpu.make_async_copy(dummy, pbufs[b], gsems[b]).wait()

    def issue_write(it, b):
        off = pl.multiple_of(it * CHUNK, CHUNK)
        pltpu.async_copy(wbufs[b], out_hbm.at[pl.ds(base + off, CHUNK)], wsems[b])

    def wait_write(b):
        pltpu.make_async_copy(wbufs[b], out_hbm.at[pl.ds(0, CHUNK)], wsems[b]).wait()

    def compute(b):
        wbuf, pbuf = wbufs[b], pbufs[b]

        def row_body(r, c2):
            for u in range(UNROLL):
                i = r * UNROLL + u
                for j in range(DIM // 16):
                    s = pl.ds(j * 16, 16)
                    wbuf[i, s] = (wbuf[i, s] + pbuf[i, s]) * 8.0
            return c2

        lax.fori_loop(0, CHUNK // UNROLL, row_body, 0)

    issue_gather(0, 0)

    def pair_body(gi, carry):
        for b in range(2):
            it = gi * 2 + b
            nb = 1 - b
            if b == 0:
                @pl.when(gi >= 1)
                def _():
                    wait_write(nb)
                issue_gather(it + 1, nb)
            else:
                wait_write(nb)

                @pl.when(gi + 1 < NCHUNKS // 2)
                def _():
                    issue_gather(it + 1, nb)
            wait_gather(b)
            compute(b)
            issue_write(it, b)
        return carry

    lax.fori_loop(0, NCHUNKS // 2, pair_body, 0)
    wait_write(1)


def kernel(tokens, token_types, word_table, type_table):
    widx = tokens.reshape(NUM_TOK).astype(jnp.int32)
    pos = jnp.arange(SEQ, dtype=jnp.int32)
    pidx = (token_types.astype(jnp.int32) * SEQ + pos[None, :]).reshape(NUM_TOK)
    pe = _positional_encoding(SEQ, DIM)
    pet = (pe[None, :, :] / 8.0 + type_table[:, None, :]).reshape(2 * SEQ, DIM)
    pet = pet.astype(jnp.float32)
    out = _emb_kernel(widx, pidx, word_table, pet)
    return out.reshape(BATCH, SEQ, DIM)


# speedup vs baseline: 7.6924x; 1.4027x over previous
import functools

import jax
import jax.numpy as jnp
from jax import lax
from jax.experimental import pallas as pl
from jax.experimental.pallas import tpu as pltpu
from jax.experimental.pallas import tpu_sc as plsc

VOCAB = 100000
DIM = 64
PDIM = 128
BATCH = 4096
SEQ = 200
NUM_TOK = BATCH * SEQ
NUM_WORKERS = 32
ROWS_W = BATCH // NUM_WORKERS
PER_W = NUM_TOK // NUM_WORKERS
GHALF = 128
VCHUNK = 200
NVCHUNKS = VOCAB // VCHUNK
UNROLL = 8

_mesh = plsc.VectorSubcoreMesh(core_axis_name="c", subcore_axis_name="s")


def _positional_encoding(token_length, embedding_dim):
    pos = jnp.arange(token_length, dtype=jnp.float32)[:, None]
    i = jnp.arange(embedding_dim)[None, :]
    angle_rates = 1.0 / jnp.power(
        10000.0, (2 * (i // 2)).astype(jnp.float32) / embedding_dim)
    angles = pos * angle_rates
    return jnp.where(i % 2 == 0, jnp.sin(angles), jnp.cos(angles)).astype(jnp.float32)


@functools.partial(
    pl.kernel,
    mesh=_mesh,
    out_type=jax.ShapeDtypeStruct((2 * VOCAB, PDIM), jnp.float32),
    scratch_types=[
        pltpu.VMEM((2, DIM), jnp.float32),
        pltpu.VMEM((VCHUNK, DIM), jnp.float32),
        pltpu.VMEM((VCHUNK, PDIM), jnp.float32),
    ],
    compiler_params=pltpu.CompilerParams(use_tc_tiling_on_sc=True),
)
def _prep_kernel(word_hbm, type_hbm, vocab2_hbm, ty_v, wch, vch):
    wid = lax.axis_index("s") * 2 + lax.axis_index("c")
    pltpu.sync_copy(type_hbm, ty_v)

    def chunk_body(j, carry):
        ci = j * NUM_WORKERS + wid

        @pl.when(ci < NVCHUNKS)
        def _():
            r0 = pl.multiple_of(ci * VCHUNK, 8)
            pltpu.sync_copy(word_hbm.at[pl.ds(r0, VCHUNK)], wch)
            for k in range(2):
                def row_body(i, c2):
                    for jj in range(DIM // 16):
                        s = pl.ds(jj * 16, 16)
                        v = (wch[i, s] + ty_v[k, s]) * 8.0
                        vch[i, s] = v
                        vch[i, pl.ds(DIM + jj * 16, 16)] = v
                    return c2

                lax.fori_loop(0, VCHUNK, row_body, 0)
                pltpu.sync_copy(
                    vch, vocab2_hbm.at[pl.ds(pl.multiple_of(k * VOCAB + r0, 8),
                                             VCHUNK)])
        return carry

    lax.fori_loop(0, (NVCHUNKS + NUM_WORKERS - 1) // NUM_WORKERS,
                  chunk_body, 0)


@functools.partial(
    pl.kernel,
    mesh=_mesh,
    out_type=jax.ShapeDtypeStruct((NUM_TOK, DIM), jnp.float32),
    scratch_types=[
        pltpu.VMEM((PER_W,), jnp.int32),
        pltpu.VMEM((SEQ, DIM), jnp.float32),
        pltpu.VMEM((SEQ, PDIM), jnp.float32),
        pltpu.VMEM((SEQ, PDIM), jnp.float32),
        pltpu.VMEM((SEQ, DIM), jnp.float32),
        pltpu.SemaphoreType.DMA,
        pltpu.SemaphoreType.DMA,
        pltpu.SemaphoreType.DMA,
        pltpu.SemaphoreType.DMA,
    ],
    compiler_params=pltpu.CompilerParams(use_tc_tiling_on_sc=True),
)
def _emb_kernel(idx_hbm, pe_hbm, vocab2_hbm, out_hbm,
                idx_v, pe_v, w0, w1, obuf, g0, g1, wr0, wr1):
    wid = lax.axis_index("s") * 2 + lax.axis_index("c")
    base = pl.multiple_of(wid * PER_W, PER_W)
    pltpu.sync_copy(idx_hbm.at[pl.ds(base, PER_W)], idx_v)
    pltpu.sync_copy(pe_hbm, pe_v)

    wbufs, gsems, wsems = (w0, w1), (g0, g1), (wr0, wr1)

    def issue_gather(it, b):
        off = pl.multiple_of(it * SEQ, 8)
        pltpu.async_copy(vocab2_hbm.at[idx_v.at[pl.ds(off, GHALF)]],
                         wbufs[b].at[pl.ds(0, GHALF)], gsems[b])
        pltpu.async_copy(vocab2_hbm.at[idx_v.at[pl.ds(off + GHALF, SEQ - GHALF)]],
                         wbufs[b].at[pl.ds(GHALF, SEQ - GHALF)], gsems[b])

    def wait_gather(b):
        pltpu.make_async_copy(vocab2_hbm.at[pl.ds(0, SEQ)], wbufs[b],
                              gsems[b]).wait()

    def issue_write(it, b):
        off = pl.multiple_of(it * SEQ, 8)
        pltpu.async_copy(obuf, out_hbm.at[pl.ds(base + off, SEQ)], wsems[b])

    def wait_write(b):
        pltpu.make_async_copy(obuf, out_hbm.at[pl.ds(0, SEQ)],
                              wsems[b]).wait()

    def compute(b):
        wbuf = wbufs[b]

        def row_body(r, c2):
            for u in range(UNROLL):
                i = r * UNROLL + u
                for j in range(DIM // 16):
                    s = pl.ds(j * 16, 16)
                    obuf[i, s] = wbuf[i, s] + pe_v[i, s]
            return c2

        lax.fori_loop(0, SEQ // UNROLL, row_body, 0)

    issue_gather(0, 0)

    def pair_body(gi, carry):
        for b in range(2):
            it = gi * 2 + b
            nb = 1 - b
            if b == 0:
                @pl.when(gi >= 1)
                def _():
                    wait_write(nb)
                issue_gather(it + 1, nb)
            else:
                wait_write(nb)

                @pl.when(gi + 1 < ROWS_W // 2)
                def _():
                    issue_gather(it + 1, nb)
            wait_gather(b)
            compute(b)
            issue_write(it, b)
        return carry

    lax.fori_loop(0, ROWS_W // 2, pair_body, 0)
    wait_write(1)


def kernel(tokens, token_types, word_table, type_table):
    idx = (tokens.astype(jnp.int32)
           + VOCAB * token_types.astype(jnp.int32)).reshape(NUM_TOK)
    pe = _positional_encoding(SEQ, DIM)
    vocab2 = _prep_kernel(word_table, type_table)
    out = _emb_kernel(idx, pe, vocab2)
    return out.reshape(BATCH, SEQ, DIM)
